# baseline (speedup 1.0000x reference)
import functools

import jax
import jax.numpy as jnp
from jax import lax
from jax.experimental import pallas as pl
from jax.experimental.pallas import tpu as pltpu
from jax.experimental.pallas import tpu_sc as plsc

NC, NS, L = 2, 16, 16
NW = NC * NS


def _logits_block(x_ref, w_ref, b_ref, lt_ref):
    lt = lax.dot_general(w_ref[...], x_ref[...], (((1,), (1,)), ((), ())),
                         preferred_element_type=jnp.float32)
    lt_ref[...] = (lt + b_ref[...])[None]


def _tc_logits(xf, W, b, blk):
    tokens, C = xf.shape
    E = W.shape[0]
    return pl.pallas_call(
        _logits_block,
        grid=(tokens // blk,),
        in_specs=[
            pl.BlockSpec((blk, C), lambda i: (i, 0)),
            pl.BlockSpec((E, C), lambda i: (0, 0)),
            pl.BlockSpec((E, 1), lambda i: (0, 0)),
        ],
        out_specs=pl.BlockSpec((1, E, blk), lambda i: (i, 0, 0)),
        out_shape=jax.ShapeDtypeStruct((tokens // blk, E, blk), jnp.float32),
    )(xf, W, b.reshape(E, 1))


def _sc_route(lt3, num_experts):
    nw, E, chunk = lt3.shape
    mesh = plsc.VectorSubcoreMesh(core_axis_name="c", subcore_axis_name="s")
    shape = jax.ShapeDtypeStruct(lt3.shape, jnp.float32)

    @functools.partial(
        pl.kernel, mesh=mesh,
        out_type=[shape, shape],
        scratch_types=[
            pltpu.VMEM((E, chunk), jnp.float32),
            pltpu.VMEM((E, chunk), jnp.float32),
            pltpu.VMEM((E, chunk), jnp.float32),
        ],
    )
    def sc_kernel(lt_hbm, r_hbm, m_hbm, in_v, r_v, m_v):
        wid = lax.axis_index("s") * NC + lax.axis_index("c")
        pltpu.sync_copy(lt_hbm.at[wid], in_v)

        @pl.loop(0, chunk, step=L)
        def _(c):
            sl = pl.ds(c, L)
            m1 = in_v[0, sl]
            i1 = jnp.zeros((L,), jnp.int32)
            m2 = jnp.full((L,), -jnp.inf, jnp.float32)
            i2 = jnp.zeros((L,), jnp.int32)
            for e in range(1, num_experts):
                v = in_v[e, sl]
                e_vec = jnp.full((L,), e, jnp.int32)
                i2n = jnp.where(v > m2, e_vec, i2)
                m2n = jnp.where(v > m2, v, m2)
                i2 = jnp.where(v > m1, i1, i2n)
                m2 = jnp.where(v > m1, m1, m2n)
                i1 = jnp.where(v > m1, e_vec, i1)
                m1 = jnp.where(v > m1, v, m1)
            z = jnp.zeros((L,), jnp.float32)
            for e in range(num_experts):
                z = z + jnp.exp(in_v[e, sl] - m1)
            a = 1.0 / (1.0 + jnp.exp((jnp.exp(m2 - m1) - 1.0) / z))
            ones = jnp.ones((L,), jnp.float32)
            zeros = jnp.zeros((L,), jnp.float32)
            for e in range(num_experts):
                sel1 = i1 == e
                sel2 = i2 == e
                m_v[e, sl] = jnp.where(sel1, ones, jnp.where(sel2, ones, zeros))
                r_v[e, sl] = jnp.where(sel1, a, jnp.where(sel2, 1.0 - a, zeros))

        pltpu.sync_copy(r_v, r_hbm.at[wid])
        pltpu.sync_copy(m_v, m_hbm.at[wid])

    return sc_kernel(lt3)


def kernel(x, W, b):
    B, T, C = x.shape
    E = W.shape[0]
    tokens = B * T
    chunk = tokens // NW
    xf = x.reshape(tokens, C)
    lt3 = _tc_logits(xf, W, b, chunk)
    r3, m3 = _sc_route(lt3, E)
    out = jnp.transpose(r3, (0, 2, 1)).reshape(B, T, E)
    mask = jnp.transpose(m3, (0, 2, 1)).reshape(B, T, E)
    return out, mask
